# baseline (speedup 1.0000x reference)
import functools
import math

import jax
import jax.numpy as jnp
from jax import lax
from jax.experimental import pallas as pl
from jax.experimental.pallas import tpu as pltpu
from jax.experimental.pallas import tpu_sc as plsc

H = 64
W = 900
FOV_UP = 3.0 / 180.0 * math.pi
FOV_DOWN = -25.0 / 180.0 * math.pi
FOV = FOV_UP - FOV_DOWN
P = H * W

CHUNK = 2048
LANES = 16
UNROLL = 8


def _tc_project_body(xt_ref, idx_ref, d_ref):
    xx = xt_ref[0]
    xy = xt_ref[1]
    xz = xt_ref[2]
    depth = jnp.sqrt(xx * xx + xy * xy + xz * xz)
    yaw = -jnp.arctan2(xy, xx)
    w = jnp.clip(xz / (depth + 1e-8), -1.0, 1.0)
    pitch = 2.0 * jnp.arctan2(w, 1.0 + jnp.sqrt((1.0 + w) * (1.0 - w)))
    vf = jnp.clip((pitch - FOV_DOWN) / FOV * H, 0.0, float(H - 1))
    uf = jnp.clip(0.5 * (yaw / math.pi + 1.0) * W, 0.0, float(W - 1))
    idx_ref[...] = vf.astype(jnp.int32) * 1024 + uf.astype(jnp.int32)
    d_ref[...] = depth


def _tc_project(xt):
    _, B, N = xt.shape
    NB = 16384
    grid = (N // NB,)
    return pl.pallas_call(
        _tc_project_body,
        grid=grid,
        in_specs=[pl.BlockSpec((3, B, NB), lambda i: (0, 0, i))],
        out_specs=[
            pl.BlockSpec((B, NB), lambda i: (0, i)),
            pl.BlockSpec((B, NB), lambda i: (0, i)),
        ],
        out_shape=[
            jax.ShapeDtypeStruct((B, N), jnp.int32),
            jax.ShapeDtypeStruct((B, N), jnp.float32),
        ],
    )(xt)


def _sc_scatter_body(idx_hbm, xt_hbm, d_hbm, out_hbm,
                     im, iba, ibb, vba, vbb, sema, semb):
    N = idx_hbm.shape[1]
    nch = N // CHUNK
    wid = lax.axis_index("c") * 16 + lax.axis_index("s")
    b = wid // 4
    r = wid % 4

    zeros = jnp.zeros((LANES,), jnp.float32)
    col_slices = list(range(0, W - (W % LANES), LANES))
    tail_cols = lax.iota(jnp.int32, LANES) + (W - LANES)

    def zero_body(i, _):
        for c0 in col_slices:
            im[i, pl.ds(c0, LANES)] = zeros
        plsc.store_scatter(im, [jnp.full((LANES,), i, jnp.int32), tail_cols],
                           zeros)
        return 0

    def run_channel(src):
        def issue(off, ib, vb, sem):
            pltpu.async_copy(idx_hbm.at[b, pl.ds(off, CHUNK)], ib, sem)
            pltpu.async_copy(src(pl.ds(off, CHUNK)), vb, sem)

        def drain(ib, vb, sem):
            pltpu.make_async_copy(idx_hbm.at[b, pl.ds(0, CHUNK)], ib, sem).wait()
            pltpu.make_async_copy(src(pl.ds(0, CHUNK)), vb, sem).wait()

        def process(ib, vb):
            ngrp = CHUNK // (LANES * UNROLL)

            def load_group(g):
                vecs = []
                base = g * (LANES * UNROLL)
                for u in range(UNROLL):
                    sl = pl.ds(base + u * LANES, LANES)
                    vecs.append(ib[sl])
                    vecs.append(vb[sl])
                return tuple(vecs)

            def store_group(vecs):
                for u in range(UNROLL):
                    vu = vecs[2 * u]
                    row = lax.shift_right_logical(vu, 10)
                    col = vu & 1023
                    plsc.store_scatter(im, [row, col], vecs[2 * u + 1])

            def inner(i, carry):
                cur = load_group(i + 1)
                store_group(carry)
                return cur

            last = lax.fori_loop(0, ngrp - 1, inner, load_group(0))
            store_group(last)

        issue(0, iba, vba, sema)
        lax.fori_loop(0, H, zero_body, 0)

        def body(j2, _):
            offb = (2 * j2 + 1) * CHUNK
            issue(offb, ibb, vbb, semb)
            drain(iba, vba, sema)
            process(iba, vba)
            offa = jnp.minimum((2 * j2 + 2) * CHUNK, N - CHUNK)
            issue(offa, iba, vba, sema)
            drain(ibb, vbb, semb)
            process(ibb, vbb)
            return 0

        lax.fori_loop(0, nch // 2, body, 0)
        drain(iba, vba, sema)

    srcs = [
        (lambda sl, c=c: xt_hbm.at[c, b, sl]) for c in range(3)
    ] + [lambda sl: d_hbm.at[b, sl]]
    for rr, src in enumerate(srcs):
        @pl.when(r == rr)
        def _(src=src):
            run_channel(src)

    pltpu.sync_copy(im, out_hbm.at[b, r])

    @pl.when(r == 3)
    def _():
        ones = jnp.ones((LANES,), jnp.float32)

        def mask_body(i, _):
            for c0 in col_slices:
                sl = pl.ds(c0, LANES)
                im[i, sl] = jnp.where(im[i, sl] > 0.0, ones, zeros)
            rowv = jnp.full((LANES,), i, jnp.int32)
            tv = plsc.load_gather(im, [rowv, tail_cols])
            plsc.store_scatter(im, [rowv, tail_cols],
                               jnp.where(tv > 0.0, ones, zeros))
            return 0

        lax.fori_loop(0, H, mask_body, 0)
        pltpu.sync_copy(im, out_hbm.at[b, 4])


def _sc_scatter(idx, xt, d):
    B, N = idx.shape
    mesh = plsc.VectorSubcoreMesh(
        core_axis_name="c", subcore_axis_name="s", num_cores=2, num_subcores=16
    )
    return pl.kernel(
        _sc_scatter_body,
        out_type=jax.ShapeDtypeStruct((B, 5, H, W), jnp.float32),
        mesh=mesh,
        compiler_params=pltpu.CompilerParams(needs_layout_passes=False),
        scratch_types=[
            pltpu.VMEM((H, W), jnp.float32),
            pltpu.VMEM((CHUNK,), jnp.int32),
            pltpu.VMEM((CHUNK,), jnp.int32),
            pltpu.VMEM((CHUNK,), jnp.float32),
            pltpu.VMEM((CHUNK,), jnp.float32),
            pltpu.SemaphoreType.DMA,
            pltpu.SemaphoreType.DMA,
        ],
    )(idx, xt, d)


def kernel(x):
    B, N, _ = x.shape
    xt = jnp.transpose(x, (2, 0, 1))
    idx, d = _tc_project(xt)
    return _sc_scatter(idx, xt, d)


# speedup vs baseline: 1.1323x; 1.1323x over previous
import functools
import math

import jax
import jax.numpy as jnp
from jax import lax
from jax.experimental import pallas as pl
from jax.experimental.pallas import tpu as pltpu
from jax.experimental.pallas import tpu_sc as plsc

H = 64
W = 900
FOV_UP = 3.0 / 180.0 * math.pi
FOV_DOWN = -25.0 / 180.0 * math.pi
FOV = FOV_UP - FOV_DOWN
P = H * W

CHUNK = 4096
LANES = 16
UNROLL = 8


def _tc_project_body(xt_ref, idx_ref, d_ref):
    xx = xt_ref[0]
    xy = xt_ref[1]
    xz = xt_ref[2]
    depth = jnp.sqrt(xx * xx + xy * xy + xz * xz)
    yaw = -jnp.arctan2(xy, xx)
    w = jnp.clip(xz / (depth + 1e-8), -1.0, 1.0)
    pitch = 2.0 * jnp.arctan2(w, 1.0 + jnp.sqrt((1.0 + w) * (1.0 - w)))
    vf = jnp.clip((pitch - FOV_DOWN) / FOV * H, 0.0, float(H - 1))
    uf = jnp.clip(0.5 * (yaw / math.pi + 1.0) * W, 0.0, float(W - 1))
    idx_ref[...] = vf.astype(jnp.int32) * 1024 + uf.astype(jnp.int32)
    d_ref[...] = depth


def _tc_project(xt):
    _, B, N = xt.shape
    NB = 16384
    grid = (N // NB,)
    return pl.pallas_call(
        _tc_project_body,
        grid=grid,
        in_specs=[pl.BlockSpec((3, B, NB), lambda i: (0, 0, i))],
        out_specs=[
            pl.BlockSpec((B, NB), lambda i: (0, i)),
            pl.BlockSpec((B, NB), lambda i: (0, i)),
        ],
        out_shape=[
            jax.ShapeDtypeStruct((B, N), jnp.int32),
            jax.ShapeDtypeStruct((B, N), jnp.float32),
        ],
    )(xt)


def _sc_scatter_body(idx_hbm, xt_hbm, d_hbm, out_hbm,
                     im, iba, ibb, vba, vbb, sema, semb):
    N = idx_hbm.shape[1]
    nch = N // CHUNK
    wid = lax.axis_index("s") * 2 + lax.axis_index("c")
    b = wid // 4
    r = wid % 4

    zeros = jnp.zeros((LANES,), jnp.float32)
    col_slices = list(range(0, W - (W % LANES), LANES))
    tail_cols = lax.iota(jnp.int32, LANES) + (W - LANES)

    def zero_body(i, _):
        for c0 in col_slices:
            im[i, pl.ds(c0, LANES)] = zeros
        plsc.store_scatter(im, [jnp.full((LANES,), i, jnp.int32), tail_cols],
                           zeros)
        return 0

    lax.fori_loop(0, H, zero_body, 0)

    def run_channel(src):
        def issue(off, ib, vb, sem):
            pltpu.async_copy(idx_hbm.at[b, pl.ds(off, CHUNK)], ib, sem)
            pltpu.async_copy(src(pl.ds(off, CHUNK)), vb, sem)

        def drain(ib, vb, sem):
            pltpu.make_async_copy(idx_hbm.at[b, pl.ds(0, CHUNK)], ib, sem).wait()
            pltpu.make_async_copy(src(pl.ds(0, CHUNK)), vb, sem).wait()

        def process(ib, vb):
            ngrp = CHUNK // (LANES * UNROLL)

            def load_group(g):
                vecs = []
                base = g * (LANES * UNROLL)
                for u in range(UNROLL):
                    sl = pl.ds(base + u * LANES, LANES)
                    vecs.append(ib[sl])
                    vecs.append(vb[sl])
                return tuple(vecs)

            def store_group(vecs):
                for u in range(UNROLL):
                    vu = vecs[2 * u]
                    row = lax.shift_right_logical(vu, 10)
                    col = vu & 1023
                    plsc.store_scatter(im, [row, col], vecs[2 * u + 1])

            def inner(i, carry):
                cur = load_group(i + 1)
                store_group(carry)
                return cur

            last = lax.fori_loop(0, ngrp - 1, inner, load_group(0))
            store_group(last)

        issue(0, iba, vba, sema)

        def body(j2, _):
            offb = (2 * j2 + 1) * CHUNK
            issue(offb, ibb, vbb, semb)
            drain(iba, vba, sema)
            process(iba, vba)
            offa = jnp.minimum((2 * j2 + 2) * CHUNK, N - CHUNK)
            issue(offa, iba, vba, sema)
            drain(ibb, vbb, semb)
            process(ibb, vbb)
            return 0

        lax.fori_loop(0, nch // 2, body, 0)
        drain(iba, vba, sema)

    srcs = [
        (lambda sl, c=c: xt_hbm.at[c, b, sl]) for c in range(3)
    ] + [lambda sl: d_hbm.at[b, sl]]
    for rr, src in enumerate(srcs):
        @pl.when(r == rr)
        def _(src=src):
            run_channel(src)

    pltpu.sync_copy(im, out_hbm.at[b, r])

    @pl.when(r == 3)
    def _():
        ones = jnp.ones((LANES,), jnp.float32)

        def mask_body(i, _):
            for c0 in col_slices:
                sl = pl.ds(c0, LANES)
                im[i, sl] = jnp.where(im[i, sl] > 0.0, ones, zeros)
            rowv = jnp.full((LANES,), i, jnp.int32)
            tv = plsc.load_gather(im, [rowv, tail_cols])
            plsc.store_scatter(im, [rowv, tail_cols],
                               jnp.where(tv > 0.0, ones, zeros))
            return 0

        lax.fori_loop(0, H, mask_body, 0)
        pltpu.sync_copy(im, out_hbm.at[b, 4])


def _sc_scatter(idx, xt, d):
    B, N = idx.shape
    mesh = plsc.VectorSubcoreMesh(
        core_axis_name="c", subcore_axis_name="s", num_cores=2, num_subcores=16
    )
    return pl.kernel(
        _sc_scatter_body,
        out_type=jax.ShapeDtypeStruct((B, 5, H, W), jnp.float32),
        mesh=mesh,
        compiler_params=pltpu.CompilerParams(needs_layout_passes=False),
        scratch_types=[
            pltpu.VMEM((H, W), jnp.float32),
            pltpu.VMEM((CHUNK,), jnp.int32),
            pltpu.VMEM((CHUNK,), jnp.int32),
            pltpu.VMEM((CHUNK,), jnp.float32),
            pltpu.VMEM((CHUNK,), jnp.float32),
            pltpu.SemaphoreType.DMA,
            pltpu.SemaphoreType.DMA,
        ],
    )(idx, xt, d)


def kernel(x):
    B, N, _ = x.shape
    xt = jnp.transpose(x, (2, 0, 1))
    idx, d = _tc_project(xt)
    return _sc_scatter(idx, xt, d)


# speedup vs baseline: 1.2055x; 1.0646x over previous
import functools
import math

import jax
import jax.numpy as jnp
from jax import lax
from jax.experimental import pallas as pl
from jax.experimental.pallas import tpu as pltpu
from jax.experimental.pallas import tpu_sc as plsc

H = 64
W = 900
FOV_UP = 3.0 / 180.0 * math.pi
FOV_DOWN = -25.0 / 180.0 * math.pi
FOV = FOV_UP - FOV_DOWN
P = H * W

CHUNK = 8192
LANES = 16
UNROLL = 8


def _tc_project_body(xt_ref, idx_ref, d_ref):
    xx = xt_ref[0]
    xy = xt_ref[1]
    xz = xt_ref[2]
    depth = jnp.sqrt(xx * xx + xy * xy + xz * xz)
    yaw = -jnp.arctan2(xy, xx)
    w = jnp.clip(xz / (depth + 1e-8), -1.0, 1.0)
    pitch = 2.0 * jnp.arctan2(w, 1.0 + jnp.sqrt((1.0 + w) * (1.0 - w)))
    vf = jnp.clip((pitch - FOV_DOWN) / FOV * H, 0.0, float(H - 1))
    uf = jnp.clip(0.5 * (yaw / math.pi + 1.0) * W, 0.0, float(W - 1))
    idx_ref[...] = vf.astype(jnp.int32) * 1024 + uf.astype(jnp.int32)
    d_ref[...] = depth


def _tc_project(xt):
    _, B, N = xt.shape
    NB = 16384
    grid = (N // NB,)
    return pl.pallas_call(
        _tc_project_body,
        grid=grid,
        in_specs=[pl.BlockSpec((3, B, NB), lambda i: (0, 0, i))],
        out_specs=[
            pl.BlockSpec((B, NB), lambda i: (0, i)),
            pl.BlockSpec((B, NB), lambda i: (0, i)),
        ],
        out_shape=[
            jax.ShapeDtypeStruct((B, N), jnp.int32),
            jax.ShapeDtypeStruct((B, N), jnp.float32),
        ],
    )(xt)


def _sc_scatter_body(idx_hbm, xt_hbm, d_hbm, out_hbm,
                     im, iba, ibb, vba, vbb, sema, semb):
    N = idx_hbm.shape[1]
    nch = N // CHUNK
    wid = lax.axis_index("s") * 2 + lax.axis_index("c")
    b = wid // 4
    r = wid % 4

    zeros = jnp.zeros((LANES,), jnp.float32)
    col_slices = list(range(0, W - (W % LANES), LANES))
    tail_cols = lax.iota(jnp.int32, LANES) + (W - LANES)

    def zero_body(i, _):
        for c0 in col_slices:
            im[i, pl.ds(c0, LANES)] = zeros
        plsc.store_scatter(im, [jnp.full((LANES,), i, jnp.int32), tail_cols],
                           zeros)
        return 0

    lax.fori_loop(0, H, zero_body, 0)

    def run_channel(src):
        def issue(off, ib, vb, sem):
            pltpu.async_copy(idx_hbm.at[b, pl.ds(off, CHUNK)], ib, sem)
            pltpu.async_copy(src(pl.ds(off, CHUNK)), vb, sem)

        def drain(ib, vb, sem):
            pltpu.make_async_copy(idx_hbm.at[b, pl.ds(0, CHUNK)], ib, sem).wait()
            pltpu.make_async_copy(src(pl.ds(0, CHUNK)), vb, sem).wait()

        def process(ib, vb):
            ngrp = CHUNK // (LANES * UNROLL)

            def load_group(g):
                vecs = []
                base = g * (LANES * UNROLL)
                for u in range(UNROLL):
                    sl = pl.ds(base + u * LANES, LANES)
                    vecs.append(ib[sl])
                    vecs.append(vb[sl])
                return tuple(vecs)

            def store_group(vecs):
                for u in range(UNROLL):
                    vu = vecs[2 * u]
                    row = lax.shift_right_logical(vu, 10)
                    col = vu & 1023
                    plsc.store_scatter(im, [row, col], vecs[2 * u + 1])

            def inner(i, carry):
                cur = load_group(i + 1)
                store_group(carry)
                return cur

            last = lax.fori_loop(0, ngrp - 1, inner, load_group(0))
            store_group(last)

        issue(0, iba, vba, sema)

        def body(j2, _):
            offb = (2 * j2 + 1) * CHUNK
            issue(offb, ibb, vbb, semb)
            drain(iba, vba, sema)
            process(iba, vba)
            offa = jnp.minimum((2 * j2 + 2) * CHUNK, N - CHUNK)
            issue(offa, iba, vba, sema)
            drain(ibb, vbb, semb)
            process(ibb, vbb)
            return 0

        lax.fori_loop(0, nch // 2, body, 0)
        drain(iba, vba, sema)

    srcs = [
        (lambda sl, c=c: xt_hbm.at[c, b, sl]) for c in range(3)
    ] + [lambda sl: d_hbm.at[b, sl]]
    for rr, src in enumerate(srcs):
        @pl.when(r == rr)
        def _(src=src):
            run_channel(src)

    pltpu.sync_copy(im, out_hbm.at[b, r])

    @pl.when(r == 3)
    def _():
        ones = jnp.ones((LANES,), jnp.float32)

        def mask_body(i, _):
            for c0 in col_slices:
                sl = pl.ds(c0, LANES)
                im[i, sl] = jnp.where(im[i, sl] > 0.0, ones, zeros)
            rowv = jnp.full((LANES,), i, jnp.int32)
            tv = plsc.load_gather(im, [rowv, tail_cols])
            plsc.store_scatter(im, [rowv, tail_cols],
                               jnp.where(tv > 0.0, ones, zeros))
            return 0

        lax.fori_loop(0, H, mask_body, 0)
        pltpu.sync_copy(im, out_hbm.at[b, 4])


def _sc_scatter(idx, xt, d):
    B, N = idx.shape
    mesh = plsc.VectorSubcoreMesh(
        core_axis_name="c", subcore_axis_name="s", num_cores=2, num_subcores=16
    )
    return pl.kernel(
        _sc_scatter_body,
        out_type=jax.ShapeDtypeStruct((B, 5, H, W), jnp.float32),
        mesh=mesh,
        compiler_params=pltpu.CompilerParams(needs_layout_passes=False),
        scratch_types=[
            pltpu.VMEM((H, W), jnp.float32),
            pltpu.VMEM((CHUNK,), jnp.int32),
            pltpu.VMEM((CHUNK,), jnp.int32),
            pltpu.VMEM((CHUNK,), jnp.float32),
            pltpu.VMEM((CHUNK,), jnp.float32),
            pltpu.SemaphoreType.DMA,
            pltpu.SemaphoreType.DMA,
        ],
    )(idx, xt, d)


def kernel(x):
    B, N, _ = x.shape
    xt = jnp.transpose(x, (2, 0, 1))
    idx, d = _tc_project(xt)
    return _sc_scatter(idx, xt, d)


# speedup vs baseline: 1.2168x; 1.0094x over previous
import functools
import math

import jax
import jax.numpy as jnp
from jax import lax
from jax.experimental import pallas as pl
from jax.experimental.pallas import tpu as pltpu
from jax.experimental.pallas import tpu_sc as plsc

H = 64
W = 900
FOV_UP = 3.0 / 180.0 * math.pi
FOV_DOWN = -25.0 / 180.0 * math.pi
FOV = FOV_UP - FOV_DOWN
P = H * W

CHUNK = 8192
LANES = 16
UNROLL = 8


def _tc_project_body(xt_ref, idx_ref, d_ref):
    xx = xt_ref[0]
    xy = xt_ref[1]
    xz = xt_ref[2]
    depth = jnp.sqrt(xx * xx + xy * xy + xz * xz)
    yaw = -jnp.arctan2(xy, xx)
    w = jnp.clip(xz / (depth + 1e-8), -1.0, 1.0)
    pitch = 2.0 * jnp.arctan2(w, 1.0 + jnp.sqrt((1.0 + w) * (1.0 - w)))
    vf = jnp.clip((pitch - FOV_DOWN) / FOV * H, 0.0, float(H - 1))
    uf = jnp.clip(0.5 * (yaw / math.pi + 1.0) * W, 0.0, float(W - 1))
    idx_ref[...] = vf.astype(jnp.int32) * 1024 + uf.astype(jnp.int32)
    d_ref[...] = depth


def _tc_project(xt):
    _, B, N = xt.shape
    NB = 16384
    grid = (N // NB,)
    return pl.pallas_call(
        _tc_project_body,
        grid=grid,
        in_specs=[pl.BlockSpec((3, B, NB), lambda i: (0, 0, i))],
        out_specs=[
            pl.BlockSpec((B, NB), lambda i: (0, i)),
            pl.BlockSpec((B, NB), lambda i: (0, i)),
        ],
        out_shape=[
            jax.ShapeDtypeStruct((B, N), jnp.int32),
            jax.ShapeDtypeStruct((B, N), jnp.float32),
        ],
    )(xt)


def _sc_scatter_body(idx_hbm, xt_hbm, d_hbm, out_hbm,
                     im, iba, ibb, vba, vbb, sema, semb):
    N = idx_hbm.shape[1]
    nch = N // CHUNK
    wid = lax.axis_index("s") * 2 + lax.axis_index("c")
    b = wid // 4
    r = wid % 4

    zeros = jnp.zeros((LANES,), jnp.float32)
    col_slices = list(range(0, W - (W % LANES), LANES))
    tail_cols = lax.iota(jnp.int32, LANES) + (W - LANES)

    def zero_body(i, _):
        for c0 in col_slices:
            im[i, pl.ds(c0, LANES)] = zeros
        plsc.store_scatter(im, [jnp.full((LANES,), i, jnp.int32), tail_cols],
                           zeros)
        return 0

    def run_channel(src):
        def issue(off, ib, vb, sem):
            pltpu.async_copy(idx_hbm.at[b, pl.ds(off, CHUNK)], ib, sem)
            pltpu.async_copy(src(pl.ds(off, CHUNK)), vb, sem)

        def drain(ib, vb, sem):
            pltpu.make_async_copy(idx_hbm.at[b, pl.ds(0, CHUNK)], ib, sem).wait()
            pltpu.make_async_copy(src(pl.ds(0, CHUNK)), vb, sem).wait()

        def process(ib, vb):
            ngrp = CHUNK // (LANES * UNROLL)

            def load_group(g):
                vecs = []
                base = g * (LANES * UNROLL)
                for u in range(UNROLL):
                    sl = pl.ds(base + u * LANES, LANES)
                    vecs.append(ib[sl])
                    vecs.append(vb[sl])
                return tuple(vecs)

            def store_group(vecs):
                for u in range(UNROLL):
                    vu = vecs[2 * u]
                    row = lax.shift_right_logical(vu, 10)
                    col = vu & 1023
                    plsc.store_scatter(im, [row, col], vecs[2 * u + 1])

            def inner(i, carry):
                cur = load_group(i + 1)
                store_group(carry)
                return cur

            last = lax.fori_loop(0, ngrp - 1, inner, load_group(0))
            store_group(last)

        issue(0, iba, vba, sema)
        lax.fori_loop(0, H, zero_body, 0)

        def body(j2, _):
            offb = (2 * j2 + 1) * CHUNK
            issue(offb, ibb, vbb, semb)
            drain(iba, vba, sema)
            process(iba, vba)
            offa = jnp.minimum((2 * j2 + 2) * CHUNK, N - CHUNK)
            issue(offa, iba, vba, sema)
            drain(ibb, vbb, semb)
            process(ibb, vbb)
            return 0

        lax.fori_loop(0, nch // 2, body, 0)
        drain(iba, vba, sema)

    srcs = [
        (lambda sl, c=c: xt_hbm.at[c, b, sl]) for c in range(3)
    ] + [lambda sl: d_hbm.at[b, sl]]
    for rr, src in enumerate(srcs):
        @pl.when(r == rr)
        def _(src=src):
            run_channel(src)

    pltpu.sync_copy(im, out_hbm.at[b, r])

    @pl.when(r == 3)
    def _():
        ones = jnp.ones((LANES,), jnp.float32)

        def mask_body(i, _):
            for c0 in col_slices:
                sl = pl.ds(c0, LANES)
                im[i, sl] = jnp.where(im[i, sl] > 0.0, ones, zeros)
            rowv = jnp.full((LANES,), i, jnp.int32)
            tv = plsc.load_gather(im, [rowv, tail_cols])
            plsc.store_scatter(im, [rowv, tail_cols],
                               jnp.where(tv > 0.0, ones, zeros))
            return 0

        lax.fori_loop(0, H, mask_body, 0)
        pltpu.sync_copy(im, out_hbm.at[b, 4])


def _sc_scatter(idx, xt, d):
    B, N = idx.shape
    mesh = plsc.VectorSubcoreMesh(
        core_axis_name="c", subcore_axis_name="s", num_cores=2, num_subcores=16
    )
    return pl.kernel(
        _sc_scatter_body,
        out_type=jax.ShapeDtypeStruct((B, 5, H, W), jnp.float32),
        mesh=mesh,
        compiler_params=pltpu.CompilerParams(needs_layout_passes=False),
        scratch_types=[
            pltpu.VMEM((H, W), jnp.float32),
            pltpu.VMEM((CHUNK,), jnp.int32),
            pltpu.VMEM((CHUNK,), jnp.int32),
            pltpu.VMEM((CHUNK,), jnp.float32),
            pltpu.VMEM((CHUNK,), jnp.float32),
            pltpu.SemaphoreType.DMA,
            pltpu.SemaphoreType.DMA,
        ],
    )(idx, xt, d)


def kernel(x):
    B, N, _ = x.shape
    xt = jnp.transpose(x, (2, 0, 1))
    idx, d = _tc_project(xt)
    return _sc_scatter(idx, xt, d)
